# baseline (speedup 1.0000x reference)
import functools
import math

import jax
import jax.numpy as jnp
from jax import lax
from jax.experimental import pallas as pl
from jax.experimental.pallas import tpu as pltpu
from jax.experimental.pallas import tpu_sc as plsc

_NUM_CORES = 2
_NUM_SUBCORES = 16
_NUM_WORKERS = _NUM_CORES * _NUM_SUBCORES
_LANE = 128
_CHUNK_TILES = 30
_TC_BLOCK_COLS = 16384


def _make_sc_copy(n, d, dtype):
    full_tiles = n // _LANE
    per_worker_tiles = full_tiles // _NUM_WORKERS
    rem_tiles = full_tiles % _NUM_WORKERS
    main_chunks, last = divmod(per_worker_tiles, _CHUNK_TILES)
    chunk_tiles = [_CHUNK_TILES] * main_chunks + ([last] if last else [])
    mesh = plsc.VectorSubcoreMesh(
        core_axis_name="c", subcore_axis_name="s",
        num_cores=_NUM_CORES, num_subcores=_NUM_SUBCORES,
    )
    buf_cols = _CHUNK_TILES * _LANE

    @functools.partial(
        pl.kernel,
        out_type=jax.ShapeDtypeStruct((d, n), dtype),
        mesh=mesh,
        scratch_types=(
            pltpu.VMEM((d, buf_cols), dtype),
            pltpu.VMEM((d, buf_cols), dtype),
            pltpu.SemaphoreType.DMA((2,)),
            pltpu.SemaphoreType.DMA((2,)),
        ),
    )
    def sc_copy(x_hbm, o_hbm, b0, b1, sem_in, sem_out):
        wid = lax.axis_index("s") * _NUM_CORES + lax.axis_index("c")
        base = pl.multiple_of(wid * (per_worker_tiles * _LANE), _LANE)
        bufs = (b0, b1)
        tasks = []
        off = 0
        for t in chunk_tiles:
            tasks.append((off, t * _LANE))
            off += t * _LANE
        n_t = len(tasks)

        def start_in(j):
            off, cols = tasks[j]
            c = pltpu.make_async_copy(
                x_hbm.at[:, pl.ds(base + off, cols)],
                bufs[j % 2].at[:, pl.ds(0, cols)], sem_in.at[j % 2])
            c.start()
            return c

        def start_out(j):
            off, cols = tasks[j]
            c = pltpu.make_async_copy(
                bufs[j % 2].at[:, pl.ds(0, cols)],
                o_hbm.at[:, pl.ds(base + off, cols)], sem_out.at[j % 2])
            c.start()
            return c

        in_d = [None] * n_t
        out_d = [None] * n_t
        in_d[0] = start_in(0)
        for j in range(n_t):
            if j + 1 < n_t:
                if j >= 1:
                    out_d[j - 1].wait()
                in_d[j + 1] = start_in(j + 1)
            in_d[j].wait()
            out_d[j] = start_out(j)
        if n_t >= 2:
            out_d[n_t - 2].wait()
        out_d[n_t - 1].wait()

        if rem_tiles:
            rem_base = per_worker_tiles * _NUM_WORKERS * _LANE

            @pl.when(wid < rem_tiles)
            def _():
                toff = pl.multiple_of(rem_base + wid * _LANE, _LANE)
                cin = pltpu.make_async_copy(
                    x_hbm.at[:, pl.ds(toff, _LANE)],
                    b0.at[:, pl.ds(0, _LANE)], sem_in.at[0])
                cin.start()
                cin.wait()
                cout = pltpu.make_async_copy(
                    b0.at[:, pl.ds(0, _LANE)],
                    o_hbm.at[:, pl.ds(toff, _LANE)], sem_out.at[0])
                cout.start()
                cout.wait()

    return sc_copy


def _tc_body(x_ref, o_ref):
    o_ref[...] = x_ref[...]


def _tc_copy(x):
    d, n = x.shape
    grid = math.ceil(n / _TC_BLOCK_COLS)
    spec = pl.BlockSpec((d, _TC_BLOCK_COLS), lambda i: (0, i))
    return pl.pallas_call(
        _tc_body,
        grid=(grid,),
        in_specs=[spec],
        out_specs=spec,
        out_shape=jax.ShapeDtypeStruct((d, n), x.dtype),
        compiler_params=pltpu.CompilerParams(skip_device_barrier=True),
    )(x)


def kernel(user_emb, item_emb):
    n, d = user_emb.shape
    ut, it = user_emb.T, item_emb.T
    ou = _tc_copy(ut)
    oi = _tc_copy(it)
    n_main = (n // _LANE) * _LANE
    if n_main != n:
        ou = lax.dynamic_update_slice(
            ou, lax.slice(ut, (0, n_main), (d, n)), (0, n_main))
    return (ou.T, oi.T)


# speedup vs baseline: 1.2474x; 1.2474x over previous
import functools
import math

import jax
import jax.numpy as jnp
from jax import lax
from jax.experimental import pallas as pl
from jax.experimental.pallas import tpu as pltpu
from jax.experimental.pallas import tpu_sc as plsc

_NUM_CORES = 2
_NUM_SUBCORES = 16
_NUM_WORKERS = _NUM_CORES * _NUM_SUBCORES
_LANE = 128
_CHUNK_TILES = 20
_NUM_BUFS = 3
_TC_BLOCK_COLS = 65536


def _make_sc_copy(n, d, dtype):
    full_tiles = n // _LANE
    per_worker_tiles = full_tiles // _NUM_WORKERS
    rem_tiles = full_tiles % _NUM_WORKERS
    main_chunks, last = divmod(per_worker_tiles, _CHUNK_TILES)
    chunk_tiles = [_CHUNK_TILES] * main_chunks + ([last] if last else [])
    mesh = plsc.VectorSubcoreMesh(
        core_axis_name="c", subcore_axis_name="s",
        num_cores=_NUM_CORES, num_subcores=_NUM_SUBCORES,
    )
    buf_cols = _CHUNK_TILES * _LANE

    @functools.partial(
        pl.kernel,
        out_type=jax.ShapeDtypeStruct((d, n), dtype),
        mesh=mesh,
        scratch_types=(
            tuple(pltpu.VMEM((d, buf_cols), dtype) for _ in range(_NUM_BUFS)),
            pltpu.SemaphoreType.DMA((_NUM_BUFS,)),
            pltpu.SemaphoreType.DMA((_NUM_BUFS,)),
        ),
    )
    def sc_copy(x_hbm, o_hbm, bufs, sem_in, sem_out):
        wid = lax.axis_index("s") * _NUM_CORES + lax.axis_index("c")
        base = pl.multiple_of(wid * (per_worker_tiles * _LANE), _LANE)
        nb = _NUM_BUFS
        tasks = []
        off = 0
        for t in chunk_tiles:
            tasks.append((off, t * _LANE))
            off += t * _LANE
        n_t = len(tasks)

        def start_in(j):
            off, cols = tasks[j]
            c = pltpu.make_async_copy(
                x_hbm.at[:, pl.ds(base + off, cols)],
                bufs[j % nb].at[:, pl.ds(0, cols)], sem_in.at[j % nb])
            c.start()
            return c

        def start_out(j):
            off, cols = tasks[j]
            c = pltpu.make_async_copy(
                bufs[j % nb].at[:, pl.ds(0, cols)],
                o_hbm.at[:, pl.ds(base + off, cols)], sem_out.at[j % nb])
            c.start()
            return c

        in_d = [None] * n_t
        out_d = [None] * n_t
        for j in range(min(nb - 1, n_t)):
            in_d[j] = start_in(j)
        for j in range(n_t):
            k = j + nb - 1
            if k < n_t:
                if j >= 1:
                    out_d[j - 1].wait()
                in_d[k] = start_in(k)
            in_d[j].wait()
            out_d[j] = start_out(j)
        for j in range(max(0, n_t - nb), n_t):
            out_d[j].wait()

        if rem_tiles:
            rem_base = per_worker_tiles * _NUM_WORKERS * _LANE

            @pl.when(wid < rem_tiles)
            def _():
                toff = pl.multiple_of(rem_base + wid * _LANE, _LANE)
                cin = pltpu.make_async_copy(
                    x_hbm.at[:, pl.ds(toff, _LANE)],
                    bufs[0].at[:, pl.ds(0, _LANE)], sem_in.at[0])
                cin.start()
                cin.wait()
                cout = pltpu.make_async_copy(
                    bufs[0].at[:, pl.ds(0, _LANE)],
                    o_hbm.at[:, pl.ds(toff, _LANE)], sem_out.at[0])
                cout.start()
                cout.wait()

    return sc_copy


def _tc_body(x_ref, o_ref):
    o_ref[...] = x_ref[...]


def _tc_copy(x):
    d, n = x.shape
    grid = math.ceil(n / _TC_BLOCK_COLS)
    spec = pl.BlockSpec((d, _TC_BLOCK_COLS), lambda i: (0, i))
    return pl.pallas_call(
        _tc_body,
        grid=(grid,),
        in_specs=[spec],
        out_specs=spec,
        out_shape=jax.ShapeDtypeStruct((d, n), x.dtype),
        compiler_params=pltpu.CompilerParams(skip_device_barrier=True),
    )(x)


def kernel(user_emb, item_emb):
    n, d = user_emb.shape
    ut, it = user_emb.T, item_emb.T
    ou = _make_sc_copy(n, d, user_emb.dtype)(ut)
    oi = _tc_copy(it)
    n_main = (n // _LANE) * _LANE
    if n_main != n:
        ou = lax.dynamic_update_slice(
            ou, lax.slice(ut, (0, n_main), (d, n)), (0, n_main))
    return (ou.T, oi.T)


# speedup vs baseline: 1.2543x; 1.0055x over previous
import functools
import math

import jax
import jax.numpy as jnp
from jax import lax
from jax.experimental import pallas as pl
from jax.experimental.pallas import tpu as pltpu
from jax.experimental.pallas import tpu_sc as plsc

_NUM_CORES = 2
_NUM_SUBCORES = 16
_NUM_WORKERS = _NUM_CORES * _NUM_SUBCORES
_LANE = 128
_CHUNK_TILES = 20
_NUM_BUFS = 3
_TC_BLOCK_COLS = 65536


def _make_sc_copy(n, d, dtype):
    full_tiles = n // _LANE
    per_worker_tiles = full_tiles // _NUM_WORKERS
    rem_tiles = full_tiles % _NUM_WORKERS
    main_chunks, last = divmod(per_worker_tiles, _CHUNK_TILES)
    chunk_tiles = [_CHUNK_TILES] * main_chunks + ([last] if last else [])
    mesh = plsc.VectorSubcoreMesh(
        core_axis_name="c", subcore_axis_name="s",
        num_cores=_NUM_CORES, num_subcores=_NUM_SUBCORES,
    )
    buf_cols = _CHUNK_TILES * _LANE

    @functools.partial(
        pl.kernel,
        out_type=jax.ShapeDtypeStruct((d, n), dtype),
        mesh=mesh,
        scratch_types=(
            tuple(pltpu.VMEM((d, buf_cols), dtype) for _ in range(_NUM_BUFS)),
            pltpu.SemaphoreType.DMA((_NUM_BUFS,)),
            pltpu.SemaphoreType.DMA((_NUM_BUFS,)),
        ),
    )
    def sc_copy(x_hbm, o_hbm, bufs, sem_in, sem_out):
        wid = lax.axis_index("s") * _NUM_CORES + lax.axis_index("c")
        base = pl.multiple_of(wid * (per_worker_tiles * _LANE), _LANE)
        nb = _NUM_BUFS
        tasks = []
        off = 0
        for t in chunk_tiles:
            tasks.append((off, t * _LANE))
            off += t * _LANE
        n_t = len(tasks)

        def start_in(j):
            off, cols = tasks[j]
            c = pltpu.make_async_copy(
                x_hbm.at[:, pl.ds(base + off, cols)],
                bufs[j % nb].at[:, pl.ds(0, cols)], sem_in.at[j % nb])
            c.start()
            return c

        def start_out(j):
            off, cols = tasks[j]
            c = pltpu.make_async_copy(
                bufs[j % nb].at[:, pl.ds(0, cols)],
                o_hbm.at[:, pl.ds(base + off, cols)], sem_out.at[j % nb])
            c.start()
            return c

        in_d = [None] * n_t
        out_d = [None] * n_t
        for j in range(min(nb - 1, n_t)):
            in_d[j] = start_in(j)
        for j in range(n_t):
            k = j + nb - 1
            if k < n_t:
                if j >= 1:
                    out_d[j - 1].wait()
                in_d[k] = start_in(k)
            in_d[j].wait()
            out_d[j] = start_out(j)
        for j in range(max(0, n_t - nb), n_t):
            out_d[j].wait()

        if rem_tiles:
            rem_base = per_worker_tiles * _NUM_WORKERS * _LANE

            @pl.when(wid < rem_tiles)
            def _():
                toff = pl.multiple_of(rem_base + wid * _LANE, _LANE)
                cin = pltpu.make_async_copy(
                    x_hbm.at[:, pl.ds(toff, _LANE)],
                    bufs[0].at[:, pl.ds(0, _LANE)], sem_in.at[0])
                cin.start()
                cin.wait()
                cout = pltpu.make_async_copy(
                    bufs[0].at[:, pl.ds(0, _LANE)],
                    o_hbm.at[:, pl.ds(toff, _LANE)], sem_out.at[0])
                cout.start()
                cout.wait()

    return sc_copy


_TC_CHUNK_TILES = 651
_TC_NBUF = 3


def _make_tc_copy(n, d, dtype):
    full_tiles = n // _LANE
    n_chunks = math.ceil(full_tiles / _TC_CHUNK_TILES)
    chunk_tiles = [_TC_CHUNK_TILES] * (full_tiles // _TC_CHUNK_TILES)
    if full_tiles % _TC_CHUNK_TILES:
        chunk_tiles.append(full_tiles % _TC_CHUNK_TILES)
    buf_cols = _TC_CHUNK_TILES * _LANE
    nb = _TC_NBUF

    def tc_body(x_hbm, o_hbm, *rest):
        bufs, sem_in, sem_out = rest[:nb], rest[nb], rest[nb + 1]
        tasks = []
        off = 0
        for t in chunk_tiles:
            tasks.append((off, t * _LANE))
            off += t * _LANE
        n_t = len(tasks)

        def start_in(j):
            off, cols = tasks[j]
            c = pltpu.make_async_copy(
                x_hbm.at[:, pl.ds(off, cols)],
                bufs[j % nb].at[:, pl.ds(0, cols)], sem_in.at[j % nb])
            c.start()
            return c

        def start_out(j):
            off, cols = tasks[j]
            c = pltpu.make_async_copy(
                bufs[j % nb].at[:, pl.ds(0, cols)],
                o_hbm.at[:, pl.ds(off, cols)], sem_out.at[j % nb])
            c.start()
            return c

        in_d = [None] * n_t
        out_d = [None] * n_t
        for j in range(min(nb - 1, n_t)):
            in_d[j] = start_in(j)
        for j in range(n_t):
            k = j + nb - 1
            if k < n_t:
                if j >= 1:
                    out_d[j - 1].wait()
                in_d[k] = start_in(k)
            in_d[j].wait()
            out_d[j] = start_out(j)
        for j in range(max(0, n_t - nb), n_t):
            out_d[j].wait()

    return pl.pallas_call(
        tc_body,
        in_specs=[pl.BlockSpec(memory_space=pltpu.MemorySpace.HBM)],
        out_specs=pl.BlockSpec(memory_space=pltpu.MemorySpace.HBM),
        out_shape=jax.ShapeDtypeStruct((d, n), dtype),
        scratch_shapes=(
            [pltpu.VMEM((d, buf_cols), dtype) for _ in range(nb)]
            + [pltpu.SemaphoreType.DMA((nb,)), pltpu.SemaphoreType.DMA((nb,))]
        ),
        compiler_params=pltpu.CompilerParams(skip_device_barrier=True),
    )


def kernel(user_emb, item_emb):
    n, d = user_emb.shape
    ut, it = user_emb.T, item_emb.T
    ou = _make_sc_copy(n, d, user_emb.dtype)(ut)
    oi = _make_tc_copy(n, d, item_emb.dtype)(it)
    n_main = (n // _LANE) * _LANE
    if n_main != n:
        ou = lax.dynamic_update_slice(
            ou, lax.slice(ut, (0, n_main), (d, n)), (0, n_main))
        oi = lax.dynamic_update_slice(
            oi, lax.slice(it, (0, n_main), (d, n)), (0, n_main))
    return (ou.T, oi.T)


# speedup vs baseline: 1.4943x; 1.1914x over previous
import functools
import math

import jax
import jax.numpy as jnp
from jax import lax
from jax.experimental import pallas as pl
from jax.experimental.pallas import tpu as pltpu
from jax.experimental.pallas import tpu_sc as plsc

_NUM_CORES = 2
_NUM_SUBCORES = 16
_NUM_WORKERS = _NUM_CORES * _NUM_SUBCORES
_LANE = 128
_CHUNK_TILES = 20
_NUM_BUFS = 3
_TC_BLOCK_COLS = 65536


def _make_sc_copy(n, d, dtype):
    full_tiles = n // _LANE
    per_worker_tiles = full_tiles // _NUM_WORKERS
    rem_tiles = full_tiles % _NUM_WORKERS
    main_chunks, last = divmod(per_worker_tiles, _CHUNK_TILES)
    chunk_tiles = [_CHUNK_TILES] * main_chunks + ([last] if last else [])
    mesh = plsc.VectorSubcoreMesh(
        core_axis_name="c", subcore_axis_name="s",
        num_cores=_NUM_CORES, num_subcores=_NUM_SUBCORES,
    )
    buf_cols = _CHUNK_TILES * _LANE

    @functools.partial(
        pl.kernel,
        out_type=jax.ShapeDtypeStruct((d, n), dtype),
        mesh=mesh,
        scratch_types=(
            tuple(pltpu.VMEM((d, buf_cols), dtype) for _ in range(_NUM_BUFS)),
            pltpu.SemaphoreType.DMA((_NUM_BUFS,)),
            pltpu.SemaphoreType.DMA((_NUM_BUFS,)),
        ),
    )
    def sc_copy(x_hbm, o_hbm, bufs, sem_in, sem_out):
        wid = lax.axis_index("s") * _NUM_CORES + lax.axis_index("c")
        base = pl.multiple_of(wid * (per_worker_tiles * _LANE), _LANE)
        nb = _NUM_BUFS
        tasks = []
        off = 0
        for t in chunk_tiles:
            tasks.append((off, t * _LANE))
            off += t * _LANE
        n_t = len(tasks)

        def start_in(j):
            off, cols = tasks[j]
            c = pltpu.make_async_copy(
                x_hbm.at[:, pl.ds(base + off, cols)],
                bufs[j % nb].at[:, pl.ds(0, cols)], sem_in.at[j % nb])
            c.start()
            return c

        def start_out(j):
            off, cols = tasks[j]
            c = pltpu.make_async_copy(
                bufs[j % nb].at[:, pl.ds(0, cols)],
                o_hbm.at[:, pl.ds(base + off, cols)], sem_out.at[j % nb])
            c.start()
            return c

        in_d = [None] * n_t
        out_d = [None] * n_t
        for j in range(min(nb - 1, n_t)):
            in_d[j] = start_in(j)
        for j in range(n_t):
            k = j + nb - 1
            if k < n_t:
                if j >= 1:
                    out_d[j - 1].wait()
                in_d[k] = start_in(k)
            in_d[j].wait()
            out_d[j] = start_out(j)
        for j in range(max(0, n_t - nb), n_t):
            out_d[j].wait()

        if rem_tiles:
            rem_base = per_worker_tiles * _NUM_WORKERS * _LANE

            @pl.when(wid < rem_tiles)
            def _():
                toff = pl.multiple_of(rem_base + wid * _LANE, _LANE)
                cin = pltpu.make_async_copy(
                    x_hbm.at[:, pl.ds(toff, _LANE)],
                    bufs[0].at[:, pl.ds(0, _LANE)], sem_in.at[0])
                cin.start()
                cin.wait()
                cout = pltpu.make_async_copy(
                    bufs[0].at[:, pl.ds(0, _LANE)],
                    o_hbm.at[:, pl.ds(toff, _LANE)], sem_out.at[0])
                cout.start()
                cout.wait()

    return sc_copy


_TC_CHUNK_TILES = 651
_TC_NBUF = 3


def _make_tc_copy(n, d, dtype):
    full_tiles = n // _LANE
    n_chunks = math.ceil(full_tiles / _TC_CHUNK_TILES)
    chunk_tiles = [_TC_CHUNK_TILES] * (full_tiles // _TC_CHUNK_TILES)
    if full_tiles % _TC_CHUNK_TILES:
        chunk_tiles.append(full_tiles % _TC_CHUNK_TILES)
    buf_cols = _TC_CHUNK_TILES * _LANE
    nb = _TC_NBUF

    def tc_body(x_hbm, o_hbm, *rest):
        bufs, sem_in, sem_out = rest[:nb], rest[nb], rest[nb + 1]
        tasks = []
        off = 0
        for t in chunk_tiles:
            tasks.append((off, t * _LANE))
            off += t * _LANE
        n_t = len(tasks)

        def start_in(j):
            off, cols = tasks[j]
            c = pltpu.make_async_copy(
                x_hbm.at[:, pl.ds(off, cols)],
                bufs[j % nb].at[:, pl.ds(0, cols)], sem_in.at[j % nb])
            c.start()
            return c

        def start_out(j):
            off, cols = tasks[j]
            c = pltpu.make_async_copy(
                bufs[j % nb].at[:, pl.ds(0, cols)],
                o_hbm.at[:, pl.ds(off, cols)], sem_out.at[j % nb])
            c.start()
            return c

        in_d = [None] * n_t
        out_d = [None] * n_t
        for j in range(min(nb - 1, n_t)):
            in_d[j] = start_in(j)
        for j in range(n_t):
            k = j + nb - 1
            if k < n_t:
                if j >= 1:
                    out_d[j - 1].wait()
                in_d[k] = start_in(k)
            in_d[j].wait()
            out_d[j] = start_out(j)
        for j in range(max(0, n_t - nb), n_t):
            out_d[j].wait()

    return pl.pallas_call(
        tc_body,
        in_specs=[pl.BlockSpec(memory_space=pltpu.MemorySpace.HBM)],
        out_specs=pl.BlockSpec(memory_space=pltpu.MemorySpace.HBM),
        out_shape=jax.ShapeDtypeStruct((d, n), dtype),
        scratch_shapes=(
            [pltpu.VMEM((d, buf_cols), dtype) for _ in range(nb)]
            + [pltpu.SemaphoreType.DMA((nb,)), pltpu.SemaphoreType.DMA((nb,))]
        ),
        compiler_params=pltpu.CompilerParams(skip_device_barrier=True),
    )


def kernel(user_emb, item_emb):
    n, d = user_emb.shape
    ut, it = user_emb.T, item_emb.T
    ou = _make_tc_copy(n, d, user_emb.dtype)(ut)
    oi = _make_tc_copy(n, d, item_emb.dtype)(it)
    n_main = (n // _LANE) * _LANE
    if n_main != n:
        ou = lax.dynamic_update_slice(
            ou, lax.slice(ut, (0, n_main), (d, n)), (0, n_main))
        oi = lax.dynamic_update_slice(
            oi, lax.slice(it, (0, n_main), (d, n)), (0, n_main))
    return (ou.T, oi.T)
